# baseline (speedup 1.0000x reference)
import functools

import jax
import jax.numpy as jnp
from jax import lax
from jax.experimental import pallas as pl
from jax.experimental.pallas import tpu as pltpu
from jax.experimental.pallas import tpu_sc as plsc

H = 128
NUM_WORKERS = 32
EDGE_BLOCK = 10000


def _sc_compose_indices(x, i, j):
    E = i.shape[0]
    N = x.shape[0]
    bpw = E // NUM_WORKERS
    mesh = plsc.VectorSubcoreMesh(core_axis_name="c", subcore_axis_name="s")

    @functools.partial(
        pl.kernel,
        mesh=mesh,
        out_type=(
            jax.ShapeDtypeStruct((E,), jnp.int32),
            jax.ShapeDtypeStruct((E,), jnp.int32),
        ),
        scratch_types=[
            pltpu.VMEM((N,), jnp.int32),
            pltpu.VMEM((bpw,), jnp.int32),
            pltpu.VMEM((bpw,), jnp.int32),
            pltpu.VMEM((bpw,), jnp.int32),
            pltpu.VMEM((bpw,), jnp.int32),
        ],
        compiler_params=pltpu.CompilerParams(needs_layout_passes=False),
    )
    def compose(x_hbm, i_hbm, j_hbm, xi_hbm, xj_hbm, xv, iv, jv, oi, oj):
        wid = lax.axis_index("s") * 2 + lax.axis_index("c")
        base = wid * bpw
        pltpu.sync_copy(x_hbm, xv)
        pltpu.sync_copy(i_hbm.at[pl.ds(base, bpw)], iv)
        pltpu.sync_copy(j_hbm.at[pl.ds(base, bpw)], jv)

        def body(e, carry):
            o = pl.multiple_of(e * 400, 400)
            for u in range(25):
                s = pl.ds(o + u * 16, 16)
                oi[s] = plsc.load_gather(xv, [iv[s]])
                oj[s] = plsc.load_gather(xv, [jv[s]])
            return carry

        lax.fori_loop(0, bpw // 400, body, 0)
        pltpu.sync_copy(oi, xi_hbm.at[pl.ds(base, bpw)])
        pltpu.sync_copy(oj, xj_hbm.at[pl.ds(base, bpw)])

    return compose(x, i, j)


def _g12_body(emb_ref, w12_ref, out_ref):
    g12 = jnp.dot(emb_ref[...], w12_ref[...],
                  preferred_element_type=jnp.float32)
    out_ref[0:H] = g12[:, :H].astype(jnp.bfloat16)
    out_ref[H:2 * H] = g12[:, H:].astype(jnp.bfloat16)


def _tc_body(rbf_ref, xi_ref, xj_ref, gstack_ref, w3t_ref, w0t_ref,
             w1rt_ref, b0_ref, bl_ref, e1_ref, e2_ref):
    B = rbf_ref.shape[0]
    rbf_b = rbf_ref[...]
    r0 = jnp.dot(rbf_b, w0t_ref[...],
                 preferred_element_type=jnp.float32) + b0_ref[...]
    r0 = r0 * jax.nn.sigmoid(r0)
    t = jnp.dot(r0.astype(jnp.bfloat16), w3t_ref[...],
                preferred_element_type=jnp.float32)

    iot = lax.broadcasted_iota(jnp.int32, (H, B), 0)
    ohi = (iot == xi_ref[0]).astype(jnp.bfloat16)
    ohj = (iot == xj_ref[0]).astype(jnp.bfloat16)
    ohcat = jnp.concatenate([ohi, ohj], axis=0)
    g = lax.dot_general(ohcat, gstack_ref[...], (((0,), (0,)), ((), ())),
                        preferred_element_type=jnp.float32)

    pre = t + g + bl_ref[...]
    e1 = pre * jax.nn.sigmoid(pre)
    e2 = jnp.dot(rbf_b, w1rt_ref[...], preferred_element_type=jnp.float32) * e1
    e1_ref[...] = e1
    e2_ref[...] = e2


def _tc_mlp(rbf, xi3, xj3, emb_pad, w12, w3t, w0t, w1rt, b0, bl):
    E, R = rbf.shape
    B = EDGE_BLOCK
    nb = E // B
    gstack = pl.pallas_call(
        _g12_body,
        out_shape=jax.ShapeDtypeStruct((2 * H, H), jnp.bfloat16),
    )(emb_pad, w12)
    full = lambda shape: pl.BlockSpec(shape, lambda b: (0,) * len(shape))
    return pl.pallas_call(
        _tc_body,
        grid=(nb,),
        in_specs=[
            pl.BlockSpec((B, R), lambda b: (b, 0)),
            pl.BlockSpec((1, 1, B), lambda b: (b, 0, 0)),
            pl.BlockSpec((1, 1, B), lambda b: (b, 0, 0)),
            full((2 * H, H)),
            full((H, H)),
            full((R, H)),
            full((R, H)),
            full((1, H)),
            full((1, H)),
        ],
        out_specs=[
            pl.BlockSpec((B, H), lambda b: (b, 0)),
            pl.BlockSpec((B, H), lambda b: (b, 0)),
        ],
        out_shape=[
            jax.ShapeDtypeStruct((E, H), jnp.float32),
            jax.ShapeDtypeStruct((E, H), jnp.float32),
        ],
        compiler_params=pltpu.CompilerParams(
            dimension_semantics=("parallel",),
        ),
    )(rbf, xi3, xj3, gstack, w3t, w0t, w1rt, b0, bl)


def kernel(x, rbf, i, j, emb_w, W_rbf0, b_rbf0, W_lin, b_lin, W_rbf1):
    E, R = rbf.shape
    B = EDGE_BLOCK
    nb = E // B
    x = x.astype(jnp.int32)
    i = i.astype(jnp.int32)
    j = j.astype(jnp.int32)

    xi, xj = _sc_compose_indices(x, i, j)

    xi3 = xi.reshape(nb, 1, B)
    xj3 = xj.reshape(nb, 1, B)
    emb_pad = jnp.zeros((H, H), jnp.float32).at[: emb_w.shape[0]].set(emb_w)
    w12 = jnp.concatenate([W_lin[:, :H].T, W_lin[:, H:2 * H].T], axis=1)
    w3t = W_lin[:, 2 * H:].T.astype(jnp.bfloat16)
    w0t = W_rbf0.T
    w1rt = W_rbf1.T
    b0 = b_rbf0.reshape(1, H)
    bl = b_lin.reshape(1, H)

    e1, e2 = _tc_mlp(rbf, xi3, xj3, emb_pad, w12, w3t, w0t, w1rt, b0, bl)
    return (e1, e2)


# speedup vs baseline: 1.3259x; 1.3259x over previous
import functools

import jax
import jax.numpy as jnp
from jax import lax
from jax.experimental import pallas as pl
from jax.experimental.pallas import tpu as pltpu
from jax.experimental.pallas import tpu_sc as plsc

H = 128
NUM_WORKERS = 32
EDGE_BLOCK = 12800


def _sc_compose_indices(x, i, j):
    E = i.shape[0]
    N = x.shape[0]
    bpw = E // NUM_WORKERS
    mesh = plsc.VectorSubcoreMesh(core_axis_name="c", subcore_axis_name="s")

    @functools.partial(
        pl.kernel,
        mesh=mesh,
        out_type=(
            jax.ShapeDtypeStruct((E,), jnp.int32),
            jax.ShapeDtypeStruct((E,), jnp.int32),
        ),
        scratch_types=[
            pltpu.VMEM((N,), jnp.int32),
            pltpu.VMEM((bpw,), jnp.int32),
            pltpu.VMEM((bpw,), jnp.int32),
            pltpu.VMEM((bpw,), jnp.int32),
            pltpu.VMEM((bpw,), jnp.int32),
        ],
        compiler_params=pltpu.CompilerParams(needs_layout_passes=False),
    )
    def compose(x_hbm, i_hbm, j_hbm, xi_hbm, xj_hbm, xv, iv, jv, oi, oj):
        wid = lax.axis_index("s") * 2 + lax.axis_index("c")
        base = wid * bpw
        pltpu.sync_copy(x_hbm, xv)
        pltpu.sync_copy(i_hbm.at[pl.ds(base, bpw)], iv)
        pltpu.sync_copy(j_hbm.at[pl.ds(base, bpw)], jv)

        def body(e, carry):
            o = pl.multiple_of(e * 400, 400)
            for u in range(25):
                s = pl.ds(o + u * 16, 16)
                oi[s] = plsc.load_gather(xv, [iv[s]])
                oj[s] = plsc.load_gather(xv, [jv[s]])
            return carry

        lax.fori_loop(0, bpw // 400, body, 0)
        pltpu.sync_copy(oi, xi_hbm.at[pl.ds(base, bpw)])
        pltpu.sync_copy(oj, xj_hbm.at[pl.ds(base, bpw)])

    return compose(x, i, j)


def _g12_body(emb_ref, w12_ref, w3tb_ref, out_ref):
    g12 = jnp.dot(emb_ref[...], w12_ref[...],
                  preferred_element_type=jnp.float32)
    out_ref[0:H] = w3tb_ref[...]
    out_ref[H:2 * H] = g12[:, :H].astype(jnp.bfloat16)
    out_ref[2 * H:3 * H] = g12[:, H:].astype(jnp.bfloat16)


def _tc_body(rbft_ref, xi_ref, xj_ref, wall_ref, w0t_ref,
             w1rt_ref, b0t_ref, bl_ref, e1_ref, e2_ref):
    B = rbft_ref.shape[1]
    rbft_b = rbft_ref[...]
    r0t = lax.dot_general(w0t_ref[...], rbft_b, (((0,), (0,)), ((), ())),
                          preferred_element_type=jnp.float32) + b0t_ref[...]
    r0t = r0t * jax.nn.sigmoid(r0t)

    iot = lax.broadcasted_iota(jnp.int32, (H, B), 0)
    ohi = (iot == xi_ref[0]).astype(jnp.bfloat16)
    ohj = (iot == xj_ref[0]).astype(jnp.bfloat16)
    lhs = jnp.concatenate([r0t.astype(jnp.bfloat16), ohi, ohj], axis=0)
    pre = lax.dot_general(lhs, wall_ref[...], (((0,), (0,)), ((), ())),
                          preferred_element_type=jnp.float32) + bl_ref[...]
    e1 = pre * jax.nn.sigmoid(pre)
    e2 = lax.dot_general(rbft_b, w1rt_ref[...], (((0,), (0,)), ((), ())),
                         preferred_element_type=jnp.float32) * e1
    e1_ref[...] = e1
    e2_ref[...] = e2


def _tc_mlp(rbft, xi3, xj3, emb_pad, w12, w3tb, w0t, w1rt, b0t, bl):
    R, E = rbft.shape
    B = EDGE_BLOCK
    nb = E // B
    wall = pl.pallas_call(
        _g12_body,
        out_shape=jax.ShapeDtypeStruct((3 * H, H), jnp.bfloat16),
    )(emb_pad, w12, w3tb)
    full = lambda shape: pl.BlockSpec(shape, lambda b: (0,) * len(shape))
    return pl.pallas_call(
        _tc_body,
        grid=(nb,),
        in_specs=[
            pl.BlockSpec((R, B), lambda b: (0, b)),
            pl.BlockSpec((1, 1, B), lambda b: (b, 0, 0)),
            pl.BlockSpec((1, 1, B), lambda b: (b, 0, 0)),
            full((3 * H, H)),
            full((R, H)),
            full((R, H)),
            full((H, 1)),
            full((1, H)),
        ],
        out_specs=[
            pl.BlockSpec((B, H), lambda b: (b, 0)),
            pl.BlockSpec((B, H), lambda b: (b, 0)),
        ],
        out_shape=[
            jax.ShapeDtypeStruct((E, H), jnp.float32),
            jax.ShapeDtypeStruct((E, H), jnp.float32),
        ],
        compiler_params=pltpu.CompilerParams(
            dimension_semantics=("parallel",),
        ),
    )(rbft, xi3, xj3, wall, w0t, w1rt, b0t, bl)


def kernel(x, rbf, i, j, emb_w, W_rbf0, b_rbf0, W_lin, b_lin, W_rbf1):
    E, R = rbf.shape
    B = EDGE_BLOCK
    nb = E // B
    x = x.astype(jnp.int32)
    i = i.astype(jnp.int32)
    j = j.astype(jnp.int32)

    xi, xj = _sc_compose_indices(x, i, j)

    xi3 = xi.reshape(nb, 1, B)
    xj3 = xj.reshape(nb, 1, B)
    emb_pad = jnp.zeros((H, H), jnp.float32).at[: emb_w.shape[0]].set(emb_w)
    w12 = jnp.concatenate([W_lin[:, :H].T, W_lin[:, H:2 * H].T], axis=1)
    w3tb = W_lin[:, 2 * H:].T.astype(jnp.bfloat16)
    w0t = W_rbf0.T
    w1rt = W_rbf1.T
    b0t = b_rbf0.reshape(H, 1)
    bl = b_lin.reshape(1, H)
    rbft = rbf.T

    e1, e2 = _tc_mlp(rbft, xi3, xj3, emb_pad, w12, w3tb, w0t, w1rt, b0t, bl)
    return (e1, e2)
